# Initial kernel scaffold; baseline (speedup 1.0000x reference)
import jax
import jax.numpy as jnp
from jax.experimental import pallas as pl


def kernel(x, edge_weight, W_in, b_in, c, Wp, bp, W, b, W_out, b_out, edge_index):
    raise NotImplementedError("write your pallas kernel here")


# speedup vs baseline: 3.3569x; 3.3569x over previous
import dataclasses
import functools

import numpy as np
import jax
import jax.numpy as jnp
from jax import lax
from jax.experimental import pallas as pl
from jax.experimental.pallas import tpu as pltpu
from jax.experimental.pallas import tpu_sc as plsc

N = 10000
E = 320000
NFEAT = 128
NHID = 256
NCLASS = 40
NLAYERS = 8
GAMMA = 0.1
ALPHA = 0.1
LAMBDA = 0.5

HALF = NHID // 2
NSUB = 16
EDGES_PER_TILE = E // NSUB
CHUNK = 200
NCHUNK = EDGES_PER_TILE // CHUNK
NPAD = 10240
ROWS_PER_TILE = NPAD // NSUB
ZROWS = 128

_mesh = plsc.VectorSubcoreMesh(core_axis_name="c", subcore_axis_name="s")

_sc_params = pltpu.CompilerParams()
if "needs_layout_passes" in pltpu.CompilerParams.__dataclass_fields__:
    _sc_params = dataclasses.replace(_sc_params, needs_layout_passes=False)


@functools.partial(
    pl.kernel,
    out_type=jax.ShapeDtypeStruct((2 * NPAD, HALF), jnp.float32),
    mesh=_mesh,
    scratch_types=[
        pltpu.VMEM((CHUNK,), jnp.int32),
        pltpu.VMEM((CHUNK,), jnp.int32),
        pltpu.VMEM((CHUNK,), jnp.float32),
        pltpu.VMEM((CHUNK, HALF), jnp.float32),
        pltpu.VMEM_SHARED((NPAD, HALF), jnp.float32),
        pltpu.SemaphoreType.DMA,
    ],
    compiler_params=_sc_params,
)
def _spmm_sc(cur_hbm, src_hbm, dst_hbm, ew_hbm, out_hbm,
             src_v, dst_v, w_v, rows_v, acc, sem):
    c = lax.axis_index("c")
    s = lax.axis_index("s")

    @pl.loop(0, ZROWS)
    def _zero(i):
        for j in range(HALF // 16):
            rows_v[i, pl.ds(j * 16, 16)] = jnp.zeros((16,), jnp.float32)

    for i in range(ROWS_PER_TILE // ZROWS):
        pltpu.sync_copy(rows_v.at[pl.ds(0, ZROWS)],
                        acc.at[pl.ds(s * ROWS_PER_TILE + i * ZROWS, ZROWS)])

    plsc.subcore_barrier()

    @pl.loop(0, NCHUNK)
    def _chunk(k):
        base = s * EDGES_PER_TILE + k * CHUNK
        pltpu.sync_copy(src_hbm.at[pl.ds(c * E + base, CHUNK)], src_v)
        pltpu.sync_copy(dst_hbm.at[pl.ds(base, CHUNK)], dst_v)
        pltpu.sync_copy(ew_hbm.at[pl.ds(base, CHUNK)], w_v)
        pltpu.async_copy(cur_hbm.at[src_v], rows_v, sem).wait()

        @pl.loop(0, CHUNK)
        def _scale(e):
            e_vec = jnp.zeros((16,), jnp.int32) + e
            wb = plsc.load_gather(w_v, [e_vec])
            for j in range(HALF // 16):
                sl = (e, pl.ds(j * 16, 16))
                rows_v[sl] = rows_v[sl] * wb

        pltpu.sync_copy(rows_v, acc.at[dst_v], add=True)

    plsc.subcore_barrier()

    pltpu.sync_copy(
        acc.at[pl.ds(s * ROWS_PER_TILE, ROWS_PER_TILE)],
        out_hbm.at[pl.ds(c * NPAD + s * ROWS_PER_TILE, ROWS_PER_TILE)])


_RB = 400
_NRB = N // _RB


def _k1_body(x_ref, w_ref, b_ref, c_ref, o_ref):
    h = jnp.dot(x_ref[...], w_ref[...], preferred_element_type=jnp.float32)
    h = jnp.maximum(h + b_ref[...], 0.0)
    h0 = (1.0 - GAMMA) * h + GAMMA * c_ref[...]
    o_ref[0] = h0[:, :HALF]
    o_ref[1] = h0[:, HALF:]


def _k1(x, W_in, b_in, cvec):
    return pl.pallas_call(
        _k1_body,
        grid=(_NRB,),
        in_specs=[
            pl.BlockSpec((_RB, NFEAT), lambda i: (i, 0)),
            pl.BlockSpec((NFEAT, NHID), lambda i: (0, 0)),
            pl.BlockSpec((1, NHID), lambda i: (0, 0)),
            pl.BlockSpec((1, NHID), lambda i: (0, 0)),
        ],
        out_specs=pl.BlockSpec((2, _RB, HALF), lambda i: (0, i, 0)),
        out_shape=jax.ShapeDtypeStruct((2, NPAD, HALF), jnp.float32),
    )(x, W_in, b_in, cvec)


def _k2_body(beta, ah_ref, h0_ref, wp_ref, w_ref, bp_ref, b_ref, o_ref):
    ah = jnp.concatenate([ah_ref[0], ah_ref[1]], axis=1)
    h0 = jnp.concatenate([h0_ref[0], h0_ref[1]], axis=1)
    lin = jnp.dot(ah, wp_ref[...], preferred_element_type=jnp.float32)
    lin = lin + bp_ref[...]
    sup = (1.0 - ALPHA) * ah + ALPHA * h0
    supw = jnp.dot(sup, w_ref[...], preferred_element_type=jnp.float32)
    gc = jnp.maximum((1.0 - beta) * sup + beta * supw + b_ref[...], 0.0)
    cur = lin + gc
    o_ref[0] = cur[:, :HALF]
    o_ref[1] = cur[:, HALF:]


def _k2(ahb, h0b, Wp_l, W_l, bp_l, b_l, beta):
    return pl.pallas_call(
        functools.partial(_k2_body, beta),
        grid=(_NRB,),
        in_specs=[
            pl.BlockSpec((2, _RB, HALF), lambda i: (0, i, 0)),
            pl.BlockSpec((2, _RB, HALF), lambda i: (0, i, 0)),
            pl.BlockSpec((NHID, NHID), lambda i: (0, 0)),
            pl.BlockSpec((NHID, NHID), lambda i: (0, 0)),
            pl.BlockSpec((1, NHID), lambda i: (0, 0)),
            pl.BlockSpec((1, NHID), lambda i: (0, 0)),
        ],
        out_specs=pl.BlockSpec((2, _RB, HALF), lambda i: (0, i, 0)),
        out_shape=jax.ShapeDtypeStruct((2, NPAD, HALF), jnp.float32),
    )(ahb, h0b, Wp_l, W_l, bp_l, b_l)


def _k3_body(cur_ref, wo_ref, bo_ref, o_ref):
    cur = jnp.concatenate([cur_ref[0], cur_ref[1]], axis=1)
    o = jnp.dot(cur, wo_ref[...], preferred_element_type=jnp.float32)
    o = o + bo_ref[...]
    m = jnp.max(o, axis=1, keepdims=True)
    ex = jnp.exp(o - m)
    lse = jnp.log(jnp.sum(ex, axis=1, keepdims=True))
    o_ref[...] = o - m - lse


def _k3(curb, W_out, b_out):
    return pl.pallas_call(
        _k3_body,
        grid=(_NRB,),
        in_specs=[
            pl.BlockSpec((2, _RB, HALF), lambda i: (0, i, 0)),
            pl.BlockSpec((NHID, NCLASS), lambda i: (0, 0)),
            pl.BlockSpec((1, NCLASS), lambda i: (0, 0)),
        ],
        out_specs=pl.BlockSpec((_RB, NCLASS), lambda i: (i, 0)),
        out_shape=jax.ShapeDtypeStruct((N, NCLASS), jnp.float32),
    )(curb, W_out, b_out)


def kernel(x, edge_weight, W_in, b_in, c, Wp, bp, W, b, W_out, b_out,
           edge_index):
    dst = edge_index[0]
    src = edge_index[1]
    src2 = jnp.concatenate([src, src + NPAD])

    h0b = _k1(x, W_in, b_in.reshape(1, NHID), c.reshape(1, NHID))
    cur = h0b
    for l in range(NLAYERS):
        ah_flat = _spmm_sc(cur.reshape(2 * NPAD, HALF), src2, dst, edge_weight)
        ahb = ah_flat.reshape(2, NPAD, HALF)
        beta = float(np.log(LAMBDA / (l + 1) + 1.0))
        cur = _k2(ahb, h0b, Wp[l], W[l], bp[l].reshape(1, NHID),
                  b[l].reshape(1, NHID), beta)
    return _k3(cur, W_out, b_out.reshape(1, NCLASS))
